# baseline (speedup 1.0000x reference)
import jax
import jax.numpy as jnp
from jax import lax
from jax.experimental import pallas as pl
from jax.experimental.pallas import tpu as pltpu
from jax.experimental.pallas import tpu_sc as plsc

B = 4096
D = 64
L = 16

_info = plsc.get_sparse_core_info()
_NC, _NS = _info.num_cores, _info.num_subcores
_NW = _NC * _NS
_BPW = B // _NW


def _extract(idv, rr):
    return idv[pl.ds(rr, L)][0]


def _sc_user_body(ids, table, out, idv, ob, sem):
    w = lax.axis_index("s") * _NC + lax.axis_index("c")
    base = w * _BPW
    sl = pl.ds(base, _BPW)
    pltpu.sync_copy(ids.at[sl], idv.at[pl.ds(0, _BPW)])

    def fire(rr, carry):
        pltpu.async_copy(table.at[_extract(idv, rr)], ob.at[rr], sem)
        return carry

    def drain(rr, carry):
        pltpu.make_async_copy(table.at[0], ob.at[rr], sem).wait()
        return carry

    lax.fori_loop(0, _BPW, fire, 0)
    lax.fori_loop(0, _BPW, drain, 0)
    pltpu.sync_copy(ob, out.at[sl])


def _sc_item_cat_body(iids, cids, table, cat_tbl, out,
                      idv, idv2, obi, obc, sem, semc):
    w = lax.axis_index("s") * _NC + lax.axis_index("c")
    base = w * _BPW
    sl = pl.ds(base, _BPW)
    pltpu.sync_copy(iids.at[sl], idv.at[pl.ds(0, _BPW)])
    pltpu.sync_copy(cids.at[sl], idv2.at[pl.ds(0, _BPW)])

    def fire(rr, carry):
        pltpu.async_copy(table.at[_extract(idv, rr)], obi.at[rr], sem)
        pltpu.async_copy(cat_tbl.at[_extract(idv2, rr)], obc.at[rr], semc)
        return carry

    def drain(rr, carry):
        pltpu.make_async_copy(table.at[0], obi.at[rr], sem).wait()
        pltpu.make_async_copy(cat_tbl.at[0], obc.at[rr], semc).wait()
        return carry

    def addloop(rr, carry):
        for c in range(D // L):
            cs = pl.ds(c * L, L)
            obi[rr, cs] = obi[rr, cs] + obc[rr, cs]
        return carry

    lax.fori_loop(0, _BPW, fire, 0)
    lax.fori_loop(0, _BPW, drain, 0)
    lax.fori_loop(0, _BPW, addloop, 0)
    pltpu.sync_copy(obi, out.at[sl])


_MESH = dict(core_axis_name="c", subcore_axis_name="s")


@jax.jit
def _sc_user(ids, table):
    f = pl.kernel(
        _sc_user_body,
        out_type=jax.ShapeDtypeStruct((B, D), jnp.float32),
        mesh=plsc.VectorSubcoreMesh(**_MESH),
        scratch_types=[
            pltpu.VMEM((_BPW + L,), jnp.int32),
            pltpu.VMEM((_BPW, D), jnp.float32),
            pltpu.SemaphoreType.DMA,
        ],
        compiler_params=pltpu.CompilerParams(use_tc_tiling_on_sc=True),
    )
    return f(ids, table)


@jax.jit
def _sc_item_cat(iids, cids, table, cat_tbl):
    f = pl.kernel(
        _sc_item_cat_body,
        out_type=jax.ShapeDtypeStruct((B, D), jnp.float32),
        mesh=plsc.VectorSubcoreMesh(**_MESH),
        scratch_types=[
            pltpu.VMEM((_BPW + L,), jnp.int32),
            pltpu.VMEM((_BPW + L,), jnp.int32),
            pltpu.VMEM((_BPW, D), jnp.float32),
            pltpu.VMEM((_BPW, D), jnp.float32),
            pltpu.SemaphoreType.DMA,
            pltpu.SemaphoreType.DMA,
        ],
        compiler_params=pltpu.CompilerParams(use_tc_tiling_on_sc=True),
    )
    return f(iids, cids, table, cat_tbl)


_BN = 512


def _tc_body(ttT_ref, tagsT_ref, ipc_ref, out_ref):
    acc = jnp.dot(ttT_ref[...], tagsT_ref[...],
                  preferred_element_type=jnp.float32)
    out_ref[...] = acc + ipc_ref[...].T


@jax.jit
def _tc_matmul_add(ttT, tagsT, ipc):
    k = ttT.shape[1]
    return pl.pallas_call(
        _tc_body,
        grid=(B // _BN,),
        in_specs=[
            pl.BlockSpec((D, k), lambda i: (0, 0)),
            pl.BlockSpec((k, _BN), lambda i: (0, i)),
            pl.BlockSpec((_BN, D), lambda i: (i, 0)),
        ],
        out_specs=pl.BlockSpec((D, _BN), lambda i: (0, i)),
        out_shape=jax.ShapeDtypeStruct((D, B), jnp.float32),
        compiler_params=pltpu.CompilerParams(
            dimension_semantics=("arbitrary",),
        ),
    )(ttT, tagsT, ipc)


def kernel(user_ids, item_ids, attr_category, attr_tags,
           user_table, item_table, category_table, tags_table):
    uids = user_ids.astype(jnp.int32)
    iids = item_ids.astype(jnp.int32)
    cids = attr_category.astype(jnp.int32)
    ipc = _sc_item_cat(iids, cids, item_table, category_table)
    item_totalT = _tc_matmul_add(tags_table.T, attr_tags.T, ipc)
    user_rows = _sc_user(uids, user_table)
    return (user_rows, item_totalT.T)


# speedup vs baseline: 1.0271x; 1.0271x over previous
import jax
import jax.numpy as jnp
from jax import lax
from jax.experimental import pallas as pl
from jax.experimental.pallas import tpu as pltpu
from jax.experimental.pallas import tpu_sc as plsc

B = 4096
D = 64
L = 16

_info = plsc.get_sparse_core_info()
_NC, _NS = _info.num_cores, _info.num_subcores
_NW = _NC * _NS
_BPW = B // _NW


def _extract(idv, rr):
    return idv[pl.ds(rr, L)][0]


def _sc_user_body(ids, table, out, idv, ob, sem):
    w = lax.axis_index("s") * _NC + lax.axis_index("c")
    base = w * _BPW
    sl = pl.ds(base, _BPW)
    pltpu.sync_copy(ids.at[sl], idv.at[pl.ds(0, _BPW)])

    def fire(rr, carry):
        pltpu.async_copy(table.at[_extract(idv, rr)], ob.at[rr], sem)
        return carry

    def drain(rr, carry):
        pltpu.make_async_copy(table.at[0], ob.at[rr], sem).wait()
        return carry

    lax.fori_loop(0, _BPW, fire, 0)
    lax.fori_loop(0, _BPW, drain, 0)
    pltpu.sync_copy(ob, out.at[sl])


def _sc_item_cat_body(iids, cids, table, cat_tbl, out,
                      idv, idv2, obi, obc, sem, semc):
    w = lax.axis_index("s") * _NC + lax.axis_index("c")
    base = w * _BPW
    sl = pl.ds(base, _BPW)
    pltpu.sync_copy(iids.at[sl], idv.at[pl.ds(0, _BPW)])
    pltpu.sync_copy(cids.at[sl], idv2.at[pl.ds(0, _BPW)])

    def fire(rr, carry):
        pltpu.async_copy(table.at[_extract(idv, rr)], obi.at[rr], sem)
        pltpu.async_copy(cat_tbl.at[_extract(idv2, rr)], obc.at[rr], semc)
        return carry

    def drain(rr, carry):
        pltpu.make_async_copy(table.at[0], obi.at[rr], sem).wait()
        pltpu.make_async_copy(cat_tbl.at[0], obc.at[rr], semc).wait()
        return carry

    def addloop(rr, carry):
        for c in range(D // L):
            cs = pl.ds(c * L, L)
            obi[rr, cs] = obi[rr, cs] + obc[rr, cs]
        return carry

    lax.fori_loop(0, _BPW, fire, 0)
    lax.fori_loop(0, _BPW, drain, 0)
    lax.fori_loop(0, _BPW, addloop, 0)
    pltpu.sync_copy(obi, out.at[sl])


_MESH = dict(core_axis_name="c", subcore_axis_name="s")


@jax.jit
def _sc_user(ids, table):
    f = pl.kernel(
        _sc_user_body,
        out_type=jax.ShapeDtypeStruct((B, D), jnp.float32),
        mesh=plsc.VectorSubcoreMesh(**_MESH),
        scratch_types=[
            pltpu.VMEM((_BPW + L,), jnp.int32),
            pltpu.VMEM((_BPW, D), jnp.float32),
            pltpu.SemaphoreType.DMA,
        ],
        compiler_params=pltpu.CompilerParams(use_tc_tiling_on_sc=True),
    )
    return f(ids, table)


@jax.jit
def _sc_item_cat(iids, cids, table, cat_tbl):
    f = pl.kernel(
        _sc_item_cat_body,
        out_type=jax.ShapeDtypeStruct((B, D), jnp.float32),
        mesh=plsc.VectorSubcoreMesh(**_MESH),
        scratch_types=[
            pltpu.VMEM((_BPW + L,), jnp.int32),
            pltpu.VMEM((_BPW + L,), jnp.int32),
            pltpu.VMEM((_BPW, D), jnp.float32),
            pltpu.VMEM((_BPW, D), jnp.float32),
            pltpu.SemaphoreType.DMA,
            pltpu.SemaphoreType.DMA,
        ],
        compiler_params=pltpu.CompilerParams(use_tc_tiling_on_sc=True),
    )
    return f(iids, cids, table, cat_tbl)


_BN = 512


def _tc_body(ttT_ref, tagsT_ref, ipc_ref, user_ref, out_ref, uout_ref):
    acc = jnp.dot(ttT_ref[...], tagsT_ref[...],
                  preferred_element_type=jnp.float32)
    out_ref[...] = acc + ipc_ref[...].T
    uout_ref[...] = user_ref[...].T


@jax.jit
def _tc_matmul_add(ttT, tagsT, ipc, user_rows):
    k = ttT.shape[1]
    return pl.pallas_call(
        _tc_body,
        grid=(B // _BN,),
        in_specs=[
            pl.BlockSpec((D, k), lambda i: (0, 0)),
            pl.BlockSpec((k, _BN), lambda i: (0, i)),
            pl.BlockSpec((_BN, D), lambda i: (i, 0)),
            pl.BlockSpec((_BN, D), lambda i: (i, 0)),
        ],
        out_specs=[
            pl.BlockSpec((D, _BN), lambda i: (0, i)),
            pl.BlockSpec((D, _BN), lambda i: (0, i)),
        ],
        out_shape=[
            jax.ShapeDtypeStruct((D, B), jnp.float32),
            jax.ShapeDtypeStruct((D, B), jnp.float32),
        ],
        compiler_params=pltpu.CompilerParams(
            dimension_semantics=("arbitrary",),
        ),
    )(ttT, tagsT, ipc, user_rows)


def kernel(user_ids, item_ids, attr_category, attr_tags,
           user_table, item_table, category_table, tags_table):
    uids = user_ids.astype(jnp.int32)
    iids = item_ids.astype(jnp.int32)
    cids = attr_category.astype(jnp.int32)
    user_rows = _sc_user(uids, user_table)
    ipc = _sc_item_cat(iids, cids, item_table, category_table)
    item_totalT, user_embT = _tc_matmul_add(
        tags_table.T, attr_tags.T, ipc, user_rows)
    return (user_embT.T, item_totalT.T)
